# baseline (speedup 1.0000x reference)
import functools

import jax
import jax.numpy as jnp
from jax.experimental import pallas as pl
from jax.experimental.pallas import tpu as pltpu


def _pos_body(b, h, w, d, row_ref, col_ref, out_ref, scratch, sem):
    colT = jnp.transpose(col_ref[0:w, :])
    rowT = jnp.transpose(row_ref[0:h, :])
    scratch[0:d] = jnp.broadcast_to(colT[:, None, :], (d, h, w))
    scratch[d : 2 * d] = jnp.broadcast_to(rowT[:, :, None], (d, h, w))
    copies = [
        pltpu.make_async_copy(scratch, out_ref.at[bb], sem) for bb in range(b)
    ]
    for c in copies:
        c.start()
    for c in copies:
        c.wait()


def kernel(x, row_embed, col_embed):
    b = x.shape[0]
    h, w = x.shape[-2], x.shape[-1]
    d = row_embed.shape[-1]

    body = functools.partial(_pos_body, b, h, w, d)
    return pl.pallas_call(
        body,
        in_specs=[
            pl.BlockSpec(memory_space=pltpu.VMEM),
            pl.BlockSpec(memory_space=pltpu.VMEM),
        ],
        out_specs=pl.BlockSpec(memory_space=pl.ANY),
        out_shape=jax.ShapeDtypeStruct((b, 2 * d, h, w), jnp.float32),
        scratch_shapes=[
            pltpu.VMEM((2 * d, h, w), jnp.float32),
            pltpu.SemaphoreType.DMA,
        ],
    )(row_embed, col_embed)
